# baseline (speedup 1.0000x reference)
import functools
import math

import jax
import jax.numpy as jnp
from jax import lax
from jax.experimental import pallas as pl
from jax.experimental.pallas import tpu as pltpu
from jax.experimental.pallas import tpu_sc as plsc

_N = 10000
_S = 32
_H = 128


@functools.cache
def _make_sc_gather(T: int, D: int, dirs: int = 2, C: int = 40,
                    dtype=jnp.int32):
    M = dirs * _S * _N
    info = plsc.get_sparse_core_info()
    nc, ns = info.num_cores, info.num_subcores
    nw = nc * ns
    per_w = M // nw
    nch = per_w // C
    nch_s = _N // C
    assert per_w == dirs * _N and _N % C == 0 and C % 8 == 0
    assert nch % 2 == 0 and nch >= 8
    mesh = plsc.VectorSubcoreMesh(core_axis_name="c", subcore_axis_name="s")

    @functools.partial(
        pl.kernel,
        mesh=mesh,
        out_type=jax.ShapeDtypeStruct((dirs * _S, _N, D), dtype),
        scratch_types=[
            pltpu.VMEM((per_w,), jnp.int32),
        ] + [pltpu.VMEM((C, D), dtype) for _ in range(4)]
          + [pltpu.SemaphoreType.DMA for _ in range(8)],
    )
    def gather_k(table_hbm, idx_hbm, out_hbm, idx_v,
                 r0, r1, r2, r3, g0, g1, g2, g3, w0, w1, w2, w3):
        wid = lax.axis_index("s") * nc + lax.axis_index("c")
        base = pl.multiple_of(wid * per_w, 8)
        rows = [r0, r1, r2, r3]
        sg = [g0, g1, g2, g3]
        sw = [w0, w1, w2, w3]
        pltpu.sync_copy(idx_hbm.at[pl.ds(base, per_w)], idx_v)

        def g_desc(jj, b):
            off = pl.multiple_of(jj * C, 8)
            return pltpu.make_async_copy(
                table_hbm.at[idx_v.at[pl.ds(off, C)]], rows[b], sg[b])

        def w_desc(jj, b):
            s_idx = dirs * wid + jj // nch_s
            n_off = pl.multiple_of((jj % nch_s) * C, 8)
            return pltpu.make_async_copy(
                rows[b], out_hbm.at[s_idx, pl.ds(n_off, C)], sw[b])

        g_desc(0, 0).start()
        g_desc(1, 1).start()
        ngrp = nch // 4
        rem = nch - 4 * ngrp

        def body(j, carry):
            for b in range(4):
                jj = 4 * j + b

                @pl.when(jj >= 2)
                def _():
                    w_desc(jj - 2, (b + 2) % 4).wait()

                if rem == 0:
                    @pl.when(jj + 2 < nch)
                    def _():
                        g_desc(jj + 2, (b + 2) % 4).start()
                else:
                    g_desc(jj + 2, (b + 2) % 4).start()
                g_desc(jj, b).wait()
                w_desc(jj, b).start()
            return carry

        lax.fori_loop(0, ngrp, body, 0)
        for jj in range(4 * ngrp, nch):
            b = jj % 4
            w_desc(jj - 2, (b + 2) % 4).wait()
            if jj + 2 < nch:
                g_desc(jj + 2, (b + 2) % 4).start()
            g_desc(jj, b).wait()
            w_desc(jj, b).start()
        w_desc(nch - 2, (nch - 2) % 4).wait()
        w_desc(nch - 1, (nch - 1) % 4).wait()

    return gather_k


_HIMASK = -65536


def _unpack2(gi):
    ga = jax.lax.bitcast_convert_type(gi << 16, jnp.float32)
    gb = jax.lax.bitcast_convert_type(gi & _HIMASK, jnp.float32)
    return ga, gb


def _pack2(ha, hb):
    ua = jax.lax.bitcast_convert_type(ha, jnp.int32)
    ub = jax.lax.bitcast_convert_type(hb, jnp.int32)
    ua = ua + 0x7FFF + ((ua >> 16) & 1)
    ub = ub + 0x7FFF + ((ub >> 16) & 1)
    return jax.lax.shift_right_logical(ua, 16) | (ub & _HIMASK)


def _attn_math(self_ref, g_ref, w_ref, *, d, hh, inv, packed):
    wm = w_ref[0]
    sh = self_ref[...]
    q = jnp.dot(sh, wm[:, :d], preferred_element_type=jnp.float32)
    sp = jnp.dot(sh, wm[:, d:d + hh], preferred_element_type=jnp.float32)

    if packed:
        d2 = d // 2
        halves = lambda s: _unpack2(g_ref[s])
        qparts = (q[:, :d2], q[:, d2:])
    else:
        halves = lambda s: (g_ref[s],)
        qparts = (q,)

    dk = qparts[0].shape[1]
    col = jax.lax.broadcasted_iota(jnp.int32, (dk, _S), 1)
    sc = None
    for s in range(_S):
        gs = halves(s)
        acc = qparts[0] * gs[0]
        for qp, gp in zip(qparts[1:], gs[1:]):
            acc = acc + qp * gp
        term = jnp.dot(acc, jnp.where(col == s, inv, 0.0),
                       preferred_element_type=jnp.float32)
        sc = term if sc is None else sc + term
    m = jnp.max(sc, axis=1, keepdims=True)
    e = jnp.exp(sc - m)
    a = e / jnp.sum(e, axis=1, keepdims=True)

    aggs = [a[:, 0:1] * gp for gp in halves(0)]
    for s in range(1, _S):
        gs = halves(s)
        for k in range(len(aggs)):
            aggs[k] = aggs[k] + a[:, s:s + 1] * gs[k]
    agg = aggs[0] if len(aggs) == 1 else jnp.concatenate(aggs, axis=1)
    np_ = jnp.dot(agg, wm[:, d + hh:], preferred_element_type=jnp.float32)
    return jnp.maximum(jnp.concatenate([sp, np_], axis=1), 0.0)


def _attn0_dir(x, g_sm, w, bn: int = 200):
    d, hh = _H, _H

    def body(self_ref, g_ref, w_ref, out_ref, outp_ref):
        h = _attn_math(self_ref, g_ref, w_ref, d=d, hh=hh,
                       inv=1.0 / math.sqrt(float(d)), packed=False)
        out_ref[...] = h
        outp_ref[...] = _pack2(h[:, :hh], h[:, hh:])

    return pl.pallas_call(
        body,
        grid=(_N // bn,),
        in_specs=[
            pl.BlockSpec((bn, d), lambda i: (i, 0)),
            pl.BlockSpec((_S, bn, d), lambda i: (0, i, 0)),
            pl.BlockSpec((1, d, d + 2 * hh), lambda i: (0, 0, 0)),
        ],
        out_specs=[
            pl.BlockSpec((bn, 2 * hh), lambda i: (i, 0)),
            pl.BlockSpec((bn, hh), lambda i: (i, 0)),
        ],
        out_shape=[jax.ShapeDtypeStruct((_N, 2 * hh), jnp.float32),
                   jax.ShapeDtypeStruct((_N, hh), jnp.int32)],
    )(x, g_sm, w)


def _attn1_dir(t1, g_sm, w, bn: int = 200):
    d, hh = 2 * _H, _H

    def body(self_ref, g_ref, w_ref, out_ref):
        h = _attn_math(self_ref, g_ref, w_ref, d=d, hh=hh,
                       inv=1.0 / math.sqrt(float(d)), packed=True)
        out_ref[...] = h

    return pl.pallas_call(
        body,
        grid=(_N // bn,),
        in_specs=[
            pl.BlockSpec((bn, d), lambda i: (i, 0)),
            pl.BlockSpec((_S, bn, d // 2), lambda i: (0, i, 0)),
            pl.BlockSpec((1, d, d + 2 * hh), lambda i: (0, 0, 0)),
        ],
        out_specs=pl.BlockSpec((bn, 2 * hh), lambda i: (i, 0)),
        out_shape=jax.ShapeDtypeStruct((_N, 2 * hh), jnp.float32),
    )(t1, g_sm, w)


def kernel(x, fw_adj, bw_adj, fw_W0, fw_W1, bw_W0, bw_W1):
    fw_nb = fw_adj[:_N, :_S].astype(jnp.int32)
    bw_nb = bw_adj[:_N, :_S].astype(jnp.int32)
    fw_idx = fw_nb.T.reshape(-1)
    bw_idx = bw_nb.T.reshape(-1)

    gather0 = _make_sc_gather(_N, _H, dirs=1, dtype=jnp.float32)
    gather1 = _make_sc_gather(_N, _H, dirs=1, dtype=jnp.int32)

    g0f = gather0(x, fw_idx)
    g0b = gather0(x, bw_idx)
    t1f, t1fpk = _attn0_dir(x, g0f, fw_W0[None])
    g1f = gather1(t1fpk, fw_idx)
    t1b, t1bpk = _attn0_dir(x, g0b, bw_W0[None])
    g1b = gather1(t1bpk, bw_idx)
    h1f = _attn1_dir(t1f, g1f, fw_W1[None])
    h1b = _attn1_dir(t1b, g1b, bw_W1[None])
    return jnp.concatenate([h1f, h1b], axis=-1)


# speedup vs baseline: 1.0578x; 1.0578x over previous
import functools
import math

import jax
import jax.numpy as jnp
from jax import lax
from jax.experimental import pallas as pl
from jax.experimental.pallas import tpu as pltpu
from jax.experimental.pallas import tpu_sc as plsc

_N = 10000
_S = 32
_H = 128


@functools.cache
def _make_sc_gather(T: int, D: int, dirs: int = 2, C: int = 40,
                    dtype=jnp.int32):
    M = dirs * _S * _N
    info = plsc.get_sparse_core_info()
    nc, ns = info.num_cores, info.num_subcores
    nw = nc * ns
    per_w = M // nw
    nch = per_w // C
    nch_s = _N // C
    assert per_w == dirs * _N and _N % C == 0 and C % 8 == 0
    assert nch % 2 == 0 and nch >= 8
    mesh = plsc.VectorSubcoreMesh(core_axis_name="c", subcore_axis_name="s")

    @functools.partial(
        pl.kernel,
        mesh=mesh,
        out_type=jax.ShapeDtypeStruct((dirs * _S, _N, D), dtype),
        scratch_types=[
            pltpu.VMEM((per_w,), jnp.int32),
        ] + [pltpu.VMEM((C, D), dtype) for _ in range(4)]
          + [pltpu.SemaphoreType.DMA for _ in range(8)],
    )
    def gather_k(table_hbm, idx_hbm, out_hbm, idx_v,
                 r0, r1, r2, r3, g0, g1, g2, g3, w0, w1, w2, w3):
        wid = lax.axis_index("s") * nc + lax.axis_index("c")
        base = pl.multiple_of(wid * per_w, 8)
        rows = [r0, r1, r2, r3]
        sg = [g0, g1, g2, g3]
        sw = [w0, w1, w2, w3]
        pltpu.sync_copy(idx_hbm.at[pl.ds(base, per_w)], idx_v)

        def g_desc(jj, b):
            off = pl.multiple_of(jj * C, 8)
            return pltpu.make_async_copy(
                table_hbm.at[idx_v.at[pl.ds(off, C)]], rows[b], sg[b])

        def w_desc(jj, b):
            s_idx = dirs * wid + jj // nch_s
            n_off = pl.multiple_of((jj % nch_s) * C, 8)
            return pltpu.make_async_copy(
                rows[b], out_hbm.at[s_idx, pl.ds(n_off, C)], sw[b])

        g_desc(0, 0).start()
        g_desc(1, 1).start()
        ngrp = nch // 4
        rem = nch - 4 * ngrp

        def body(j, carry):
            for b in range(4):
                jj = 4 * j + b

                @pl.when(jj >= 2)
                def _():
                    w_desc(jj - 2, (b + 2) % 4).wait()

                if rem == 0:
                    @pl.when(jj + 2 < nch)
                    def _():
                        g_desc(jj + 2, (b + 2) % 4).start()
                else:
                    g_desc(jj + 2, (b + 2) % 4).start()
                g_desc(jj, b).wait()
                w_desc(jj, b).start()
            return carry

        lax.fori_loop(0, ngrp, body, 0)
        for jj in range(4 * ngrp, nch):
            b = jj % 4
            w_desc(jj - 2, (b + 2) % 4).wait()
            if jj + 2 < nch:
                g_desc(jj + 2, (b + 2) % 4).start()
            g_desc(jj, b).wait()
            w_desc(jj, b).start()
        w_desc(nch - 2, (nch - 2) % 4).wait()
        w_desc(nch - 1, (nch - 1) % 4).wait()

    return gather_k


_HIMASK = -65536


def _unpack2(gi):
    ga = jax.lax.bitcast_convert_type(gi << 16, jnp.float32)
    gb = jax.lax.bitcast_convert_type(gi & _HIMASK, jnp.float32)
    return ga, gb


def _pack2(ha, hb):
    ua = jax.lax.bitcast_convert_type(ha, jnp.int32)
    ub = jax.lax.bitcast_convert_type(hb, jnp.int32)
    ua = ua + 0x7FFF + ((ua >> 16) & 1)
    ub = ub + 0x7FFF + ((ub >> 16) & 1)
    return jax.lax.shift_right_logical(ua, 16) | (ub & _HIMASK)


def _attn_math(self_ref, g_ref, w_ref, *, d, hh, inv, packed):
    wm = w_ref[0]
    sh = self_ref[...]
    q = jnp.dot(sh, wm[:, :d], preferred_element_type=jnp.float32)
    sp = jnp.dot(sh, wm[:, d:d + hh], preferred_element_type=jnp.float32)

    if packed:
        d2 = d // 2
        halves = lambda s: _unpack2(g_ref[s])
        qparts = (q[:, :d2], q[:, d2:])
    else:
        halves = lambda s: (g_ref[s],)
        qparts = (q,)

    cols = []
    for s in range(_S):
        gs = halves(s)
        acc = qparts[0] * gs[0]
        for qp, gp in zip(qparts[1:], gs[1:]):
            acc = acc + qp * gp
        cols.append(jnp.sum(acc, axis=1, keepdims=True))
    sc = jnp.concatenate(cols, axis=1) * inv
    m = jnp.max(sc, axis=1, keepdims=True)
    e = jnp.exp(sc - m)
    a = e / jnp.sum(e, axis=1, keepdims=True)

    aggs = [a[:, 0:1] * gp for gp in halves(0)]
    for s in range(1, _S):
        gs = halves(s)
        for k in range(len(aggs)):
            aggs[k] = aggs[k] + a[:, s:s + 1] * gs[k]
    agg = aggs[0] if len(aggs) == 1 else jnp.concatenate(aggs, axis=1)
    np_ = jnp.dot(agg, wm[:, d + hh:], preferred_element_type=jnp.float32)
    return jnp.maximum(jnp.concatenate([sp, np_], axis=1), 0.0)


def _attn0_dir(x, g_sm, w, bn: int = 400):
    d, hh = _H, _H

    def body(self_ref, g_ref, w_ref, out_ref, outp_ref):
        h = _attn_math(self_ref, g_ref, w_ref, d=d, hh=hh,
                       inv=1.0 / math.sqrt(float(d)), packed=False)
        out_ref[...] = h
        outp_ref[...] = _pack2(h[:, :hh], h[:, hh:])

    return pl.pallas_call(
        body,
        grid=(_N // bn,),
        in_specs=[
            pl.BlockSpec((bn, d), lambda i: (i, 0)),
            pl.BlockSpec((_S, bn, d), lambda i: (0, i, 0)),
            pl.BlockSpec((1, d, d + 2 * hh), lambda i: (0, 0, 0)),
        ],
        out_specs=[
            pl.BlockSpec((bn, 2 * hh), lambda i: (i, 0)),
            pl.BlockSpec((bn, hh), lambda i: (i, 0)),
        ],
        out_shape=[jax.ShapeDtypeStruct((_N, 2 * hh), jnp.float32),
                   jax.ShapeDtypeStruct((_N, hh), jnp.int32)],
    )(x, g_sm, w)


def _attn1_dir(t1, g_sm, w, bn: int = 400):
    d, hh = 2 * _H, _H

    def body(self_ref, g_ref, w_ref, out_ref):
        h = _attn_math(self_ref, g_ref, w_ref, d=d, hh=hh,
                       inv=1.0 / math.sqrt(float(d)), packed=True)
        out_ref[...] = h

    return pl.pallas_call(
        body,
        grid=(_N // bn,),
        in_specs=[
            pl.BlockSpec((bn, d), lambda i: (i, 0)),
            pl.BlockSpec((_S, bn, d // 2), lambda i: (0, i, 0)),
            pl.BlockSpec((1, d, d + 2 * hh), lambda i: (0, 0, 0)),
        ],
        out_specs=pl.BlockSpec((bn, 2 * hh), lambda i: (i, 0)),
        out_shape=jax.ShapeDtypeStruct((_N, 2 * hh), jnp.float32),
    )(t1, g_sm, w)


def kernel(x, fw_adj, bw_adj, fw_W0, fw_W1, bw_W0, bw_W1):
    fw_nb = fw_adj[:_N, :_S].astype(jnp.int32)
    bw_nb = bw_adj[:_N, :_S].astype(jnp.int32)
    fw_idx = fw_nb.T.reshape(-1)
    bw_idx = bw_nb.T.reshape(-1)

    gather0 = _make_sc_gather(_N, _H, dirs=1, dtype=jnp.float32)
    gather1 = _make_sc_gather(_N, _H, dirs=1, dtype=jnp.int32)

    g0f = gather0(x, fw_idx)
    g0b = gather0(x, bw_idx)
    t1f, t1fpk = _attn0_dir(x, g0f, fw_W0[None])
    g1f = gather1(t1fpk, fw_idx)
    t1b, t1bpk = _attn0_dir(x, g0b, bw_W0[None])
    g1b = gather1(t1bpk, bw_idx)
    h1f = _attn1_dir(t1f, g1f, fw_W1[None])
    h1b = _attn1_dir(t1b, g1b, bw_W1[None])
    return jnp.concatenate([h1f, h1b], axis=-1)
